# Initial kernel scaffold; baseline (speedup 1.0000x reference)
import jax
import jax.numpy as jnp
from jax.experimental import pallas as pl


def kernel(x, params, edge_index, batch):
    raise NotImplementedError("write your pallas kernel here")


# speedup vs baseline: 1.0502x; 1.0502x over previous
import functools

import numpy as np

import jax
import jax.numpy as jnp
from jax import lax
from jax.experimental import pallas as pl
from jax.experimental.pallas import tpu as pltpu
from jax.experimental.pallas import tpu_sc as plsc

N_NODES = 10000
N_EDGES = 320000
D_IN = 128
HID = 256
LAT = 64
B = 128
NMAX = 50
N_PAIRS = NMAX * (NMAX - 1) // 2

LANES = 128
C_E = 2560
E_PAD = C_E * LANES
M_PAD_NODES = 10240
M_PAD_POOL = 256
C_POOL = 128

_BN = 1.0 + 1e-5


NV = 128 // 16
LBUF = 648
CPS = C_E // 32


def _fold_chunk(rows_v, dst_v, slot, bslot, j, frow, local, lbase,
                carry):
    accs, cur = carry
    for g in range(LANES // 16):
        idvec = dst_v[bslot, j, pl.ds(g * 16, 16)]
        for l in range(16):
            d = idvec[l]
            neq = d != cur

            @pl.when(neq)
            def _flush(accs=accs, cur=cur):
                for v in range(NV):
                    frow[pl.ds(v * 16, 16)] = accs[v]
                pltpu.sync_copy(
                    frow, local.at[pl.ds(lbase + cur * 128, 128)])
            row = [rows_v[slot, g * 16 + l, pl.ds(v * 16, 16)]
                   for v in range(NV)]
            accs = tuple(jnp.where(neq, row[v], accs[v] + row[v])
                         for v in range(NV))
            cur = d
    return accs, cur


def _make_segfold(edge_split, n_ranges_per_tile, lbuf, cps, linear_out,
                  C, MPAD):
    IDXB = min(16, cps)
    NBr = cps // IDXB
    NPAIR = IDXB // 2
    mesh = plsc.VectorSubcoreMesh(core_axis_name="c", subcore_axis_name="s")
    scratch = [
        pltpu.VMEM((2, IDXB, LANES), jnp.int32),
        pltpu.VMEM((2, IDXB, LANES), jnp.int32),
        pltpu.VMEM((2, LANES, 128), jnp.float32),
        pltpu.VMEM_SHARED((16 * lbuf * 128,), jnp.float32),
        pltpu.VMEM((128,), jnp.float32),
        pltpu.VMEM((1024,), jnp.float32),
        pltpu.VMEM((8, LANES), jnp.int32),
        pltpu.SemaphoreType.DMA,
        pltpu.SemaphoreType.DMA,
    ]
    out_rows = MPAD if edge_split else 2 * MPAD

    @functools.partial(
        pl.kernel,
        out_type=jax.ShapeDtypeStruct((out_rows * 128,), jnp.float32),
        mesh=mesh,
        scratch_types=scratch,
    )
    def seg(table, srcs, ldsts, rmeta, out, src_v, dst_v, rows_v, local,
            frow, zbuf, meta_v, gsem, isem):
        c = lax.axis_index("c")
        s = lax.axis_index("s")
        zvec = jnp.zeros((16,), jnp.float32)
        lbase = s * (lbuf * 128)

        def _range_body(rng, _rcarry):
            for zi in range(1024 // 16):
                zbuf[pl.ds(zi * 16, 16)] = zvec
            if edge_split:
                r = c * 16 + s
                base = r * cps
                src_base = base
            elif linear_out:
                r = s
                base = s * cps
                src_base = c * C + base
            else:
                r = 2 * s + rng
                base = r * cps
                src_base = c * C + base

            def zbody(k, carry):
                pltpu.sync_copy(zbuf,
                                local.at[pl.ds(lbase + k * 1024, 1024)])
                return carry
            lax.fori_loop(0, (lbuf * 128) // 1024, zbody, 0)
            pltpu.sync_copy(rmeta.at[pl.ds(r * 8, 8)], meta_v)

            pltpu.sync_copy(srcs.at[pl.ds(src_base, IDXB)], src_v.at[0])
            pltpu.sync_copy(ldsts.at[pl.ds(base, IDXB)], dst_v.at[0])
            pltpu.async_copy(table.at[src_v.at[0, 0]], rows_v.at[0], gsem)

            carry0 = (tuple(zvec for _ in range(NV)),
                      jnp.int32(lbuf - 8))

            def block_body(b, carry):
                bslot = lax.rem(b, 2)
                nbslot = lax.rem(b + 1, 2)

                @pl.when(b + 1 < NBr)
                def _prefetch_idx():
                    pltpu.async_copy(
                        srcs.at[pl.ds(src_base + (b + 1) * IDXB, IDXB)],
                        src_v.at[nbslot], isem)
                    pltpu.async_copy(
                        ldsts.at[pl.ds(base + (b + 1) * IDXB, IDXB)],
                        dst_v.at[nbslot], isem)

                def chunk_body(j, carry2):
                    slot = lax.rem(j, 2)
                    nslot = lax.rem(j + 1, 2)
                    pltpu.make_async_copy(
                        table.at[src_v.at[bslot, j]], rows_v.at[slot],
                        gsem).wait()

                    @pl.when(j + 1 < IDXB)
                    def _fire_next():
                        pltpu.async_copy(
                            table.at[src_v.at[bslot, j + 1]],
                            rows_v.at[nslot], gsem)
                    return _fold_chunk(rows_v, dst_v, slot, bslot, j,
                                       frow, local, lbase, carry2)
                carry = lax.fori_loop(0, IDXB, chunk_body, carry)

                @pl.when(b + 1 < NBr)
                def _next_block():
                    pltpu.make_async_copy(
                        srcs.at[pl.ds(src_base + (b + 1) * IDXB, IDXB)],
                        src_v.at[nbslot], isem).wait()
                    pltpu.make_async_copy(
                        ldsts.at[pl.ds(base + (b + 1) * IDXB, IDXB)],
                        dst_v.at[nbslot], isem).wait()
                    pltpu.async_copy(
                        table.at[src_v.at[nbslot, 0]], rows_v.at[0], gsem)
                return carry
            accs, cur = lax.fori_loop(0, NBr, block_body, carry0)
            for v in range(NV):
                frow[pl.ds(v * 16, 16)] = accs[v]
            pltpu.sync_copy(frow, local.at[pl.ds(lbase + cur * 128, 128)])

            if linear_out:
                pltpu.sync_copy(local.at[pl.ds(lbase, 1024)], zbuf)
                pltpu.sync_copy(
                    zbuf, out.at[pl.ds((c * MPAD + s * 8) * 128, 1024)])
            else:
                mvec = meta_v[0, pl.ds(0, 16)]
                start = mvec[0]
                span = mvec[1]
                ob = (start if edge_split else c * MPAD + start) * 128

                def cbody(k, carry):
                    pltpu.sync_copy(
                        local.at[pl.ds(lbase + k * 1024, 1024)], zbuf)
                    pltpu.sync_copy(zbuf,
                                    out.at[pl.ds(ob + k * 1024, 1024)])
                    return carry
                nfull = span // 8
                lax.fori_loop(0, nfull, cbody, 0)

                def tbody(j, carry):
                    o = nfull * 1024 + j * 128
                    pltpu.sync_copy(
                        local.at[pl.ds(lbase + o, 128)],
                        zbuf.at[pl.ds(0, 128)])
                    pltpu.sync_copy(zbuf.at[pl.ds(0, 128)],
                                    out.at[pl.ds(ob + o, 128)])
                    return carry
                lax.fori_loop(0, span - nfull * 8, tbody, 0)
            return _rcarry
        lax.fori_loop(0, n_ranges_per_tile, _range_body, 0)

    return seg


_seg_layer0 = _make_segfold(True, 1, LBUF, CPS, False, C_E, M_PAD_NODES)
_seg_layer = _make_segfold(False, 2, LBUF, CPS, False, C_E, M_PAD_NODES)
_seg_pool = _make_segfold(False, 1, 16, C_POOL // 16, True, C_POOL,
                          M_PAD_POOL)


def _leaky(t):
    return jnp.where(t >= 0, t, t * 0.2)


def _mlp0_body(x_ref, p_ref, w1_ref, b1_ref, w2_ref, b2_ref, o_ref):
    hin = x_ref[...] + p_ref[...]
    t = jnp.dot(hin, w1_ref[...], preferred_element_type=jnp.float32)
    t = _leaky(t + b1_ref[...])
    t = t / jnp.sqrt(_BN)
    t = jnp.dot(t, w2_ref[...], preferred_element_type=jnp.float32)
    t = _leaky(t + b2_ref[...])
    o_ref[0] = t[:, :128]
    o_ref[1] = t[:, 128:]


def _mlp_body(h_ref, a_ref, w1_ref, b1_ref, w2_ref, b2_ref, o_ref):
    hin = jnp.concatenate(
        [h_ref[0] + a_ref[0], h_ref[1] + a_ref[1]], axis=-1)
    t = jnp.dot(hin, w1_ref[...], preferred_element_type=jnp.float32)
    t = _leaky(t + b1_ref[...])
    t = t / jnp.sqrt(_BN)
    t = jnp.dot(t, w2_ref[...], preferred_element_type=jnp.float32)
    t = _leaky(t + b2_ref[...])
    o_ref[0] = t[:, :128]
    o_ref[1] = t[:, 128:]


_MLP_G = 10
_MLP_R = N_NODES // _MLP_G


def _mlp0(x, parts, W1, b1, W2, b2):
    return pl.pallas_call(
        _mlp0_body,
        grid=(_MLP_G,),
        in_specs=[
            pl.BlockSpec((_MLP_R, 128), lambda g: (g, 0)),
            pl.BlockSpec((_MLP_R, 128), lambda g: (g, 0)),
            pl.BlockSpec((D_IN, HID), lambda g: (0, 0)),
            pl.BlockSpec((1, HID), lambda g: (0, 0)),
            pl.BlockSpec((HID, HID), lambda g: (0, 0)),
            pl.BlockSpec((1, HID), lambda g: (0, 0)),
        ],
        out_specs=pl.BlockSpec((2, _MLP_R, 128), lambda g: (0, g, 0)),
        out_shape=jax.ShapeDtypeStruct((2, N_NODES, 128), jnp.float32),
    )(x, parts, W1, b1.reshape(1, HID), W2, b2.reshape(1, HID))


def _mlp(h, agg, W1, b1, W2, b2):
    return pl.pallas_call(
        _mlp_body,
        grid=(_MLP_G,),
        in_specs=[
            pl.BlockSpec((2, _MLP_R, 128), lambda g: (0, g, 0)),
            pl.BlockSpec((2, _MLP_R, 128), lambda g: (0, g, 0)),
            pl.BlockSpec((HID, HID), lambda g: (0, 0)),
            pl.BlockSpec((1, HID), lambda g: (0, 0)),
            pl.BlockSpec((HID, HID), lambda g: (0, 0)),
            pl.BlockSpec((1, HID), lambda g: (0, 0)),
        ],
        out_specs=pl.BlockSpec((2, _MLP_R, 128), lambda g: (0, g, 0)),
        out_shape=jax.ShapeDtypeStruct((2, N_NODES, 128), jnp.float32),
    )(h, agg, W1, b1.reshape(1, HID), W2, b2.reshape(1, HID))


NPAD_COLS = 1280
ADJ_COLS = 2560


def _build_scatter_mat():
    iu0, iu1 = np.triu_indices(NMAX, k=1)
    S = np.zeros((NPAD_COLS, ADJ_COLS), np.float32)
    p = np.arange(N_PAIRS)
    S[p, iu0 * NMAX + iu1] = 1.0
    S[p, iu1 * NMAX + iu0] = 1.0
    return S


_SCATTER_MAT_NP = _build_scatter_mat()


def _dec_body(p_ref, flw_ref, flb_ref, fmw_ref, fmb_ref, d0w_ref, d0b_ref,
              d1w_ref, d1b_ref, we_ref, be_ref, wo_ref, bo_ref,
              g0_ref, g1_ref, s_ref, o_ref):
    pooled = jnp.concatenate([p_ref[0], p_ref[1]], axis=-1)
    pooled = pooled / jnp.sqrt(_BN)
    zg = jnp.dot(pooled, flw_ref[...],
                 preferred_element_type=jnp.float32) + flb_ref[...]
    mu = jnp.dot(zg, fmw_ref[...],
                 preferred_element_type=jnp.float32) + fmb_ref[...]
    t = jnp.maximum(jnp.dot(mu, d0w_ref[...],
                    preferred_element_type=jnp.float32) + d0b_ref[...], 0.0)
    t = jnp.maximum(jnp.dot(t, d1w_ref[...],
                    preferred_element_type=jnp.float32) + d1b_ref[...], 0.0)
    z0 = jnp.dot(t, we_ref[...],
                 preferred_element_type=jnp.float32) + be_ref[...]
    z1 = jnp.dot(t, wo_ref[...],
                 preferred_element_type=jnp.float32) + bo_ref[...]
    zz = jnp.where(z0 + g0_ref[...] >= z1 + g1_ref[...], 1.0, 0.0)
    o_ref[...] = jnp.dot(zz.astype(jnp.bfloat16), s_ref[...],
                         preferred_element_type=jnp.float32)


def _decoder(pooled, params, g0, g1):
    def pad_cols(a, n):
        return jnp.pad(a, [(0, 0)] * (a.ndim - 1) + [(0, n - a.shape[-1])])

    We = pad_cols(params['dec2_W'][:, 0::2], NPAD_COLS)
    Wo = pad_cols(params['dec2_W'][:, 1::2], NPAD_COLS)
    be = pad_cols(params['dec2_b'][0::2].reshape(1, N_PAIRS), NPAD_COLS)
    bo = pad_cols(params['dec2_b'][1::2].reshape(1, N_PAIRS), NPAD_COLS)
    return pl.pallas_call(
        _dec_body,
        out_shape=jax.ShapeDtypeStruct((B, ADJ_COLS), jnp.float32),
    )(pooled, params['fc_latent_W'], params['fc_latent_b'].reshape(1, LAT),
      params['fc_mu_W'], params['fc_mu_b'].reshape(1, LAT),
      params['dec0_W'], params['dec0_b'].reshape(1, HID),
      params['dec1_W'], params['dec1_b'].reshape(1, HID),
      We, be, Wo, bo, pad_cols(g0, NPAD_COLS), pad_cols(g1, NPAD_COLS),
      jnp.asarray(_SCATTER_MAT_NP, dtype=jnp.bfloat16))


def _build_indices(src, dst, batch):
    order = jnp.argsort(dst, stable=True)
    sdst = dst[order]
    ssrc = src[order]
    RSLOT = CPS * LANES
    cut = sdst[jnp.arange(32, dtype=jnp.int32) * (N_EDGES // 32)]
    start_node = jnp.concatenate([
        jnp.zeros((1,), jnp.int32), cut[1:],
        jnp.full((1,), N_NODES, jnp.int32)])
    estart = jnp.searchsorted(sdst, cut, side='left').astype(jnp.int32)
    eidx = jnp.arange(N_EDGES, dtype=jnp.int32)
    erange = (jnp.searchsorted(estart, eidx, side='right') - 1).astype(
        jnp.int32)
    pos = eidx - estart[erange]
    slot = jnp.where(pos < RSLOT, erange * RSLOT + pos, E_PAD)
    ldst = sdst - start_node[erange]
    ldst = jnp.where(ldst < LBUF - 8, ldst, (LBUF - 8) + (eidx % 8))
    sidx = jnp.arange(E_PAD, dtype=jnp.int32)
    src_p = ((sidx * 131) % N_NODES).at[slot].set(ssrc, mode='drop')
    dst_p = ((LBUF - 8) + sidx % 8).at[slot].set(ldst, mode='drop')
    srcs_plain = src_p.reshape(C_E, LANES)
    srcs_off = jnp.concatenate([src_p, src_p + N_NODES]).reshape(
        2 * C_E, LANES)
    dsts = dst_p.reshape(C_E, LANES)
    sn = start_node[:32]
    span = start_node[1:] - start_node[:32]
    rmeta = jnp.zeros((256, LANES), jnp.int32)
    rmeta = rmeta.at[jnp.arange(32) * 8, 0].set(sn)
    rmeta = rmeta.at[jnp.arange(32) * 8, 1].set(span)

    CPP = C_POOL // 32
    PSLOT = CPP * LANES
    gstart = jnp.searchsorted(
        batch, jnp.arange(32, dtype=jnp.int32) * (B // 32),
        side='left').astype(jnp.int32)
    nidx = jnp.arange(N_NODES, dtype=jnp.int32)
    nrange = (jnp.searchsorted(gstart, nidx, side='right') - 1).astype(
        jnp.int32)
    npos = nidx - gstart[nrange]
    nslot = jnp.where(npos < PSLOT, nrange * PSLOT + npos,
                      C_POOL * LANES)
    qidx = jnp.arange(C_POOL * LANES, dtype=jnp.int32)
    nid = ((qidx * 41) % N_NODES).at[nslot].set(nidx, mode='drop')
    pdst = (8 + qidx % 8).at[nslot].set(batch % 8, mode='drop')
    pool_srcs = jnp.concatenate([nid, nid + N_NODES]).reshape(
        2 * C_POOL, LANES)
    pool_dsts = pdst.reshape(C_POOL, LANES)
    return srcs_plain, srcs_off, dsts, rmeta, pool_srcs, pool_dsts


def kernel(x, params, edge_index, batch):
    x = x.astype(jnp.float32)
    src = edge_index[0].astype(jnp.int32)
    dst = edge_index[1].astype(jnp.int32)
    batch = batch.astype(jnp.int32)
    srcs_plain, srcs_off, dsts, rmeta, pool_srcs, pool_dsts = (
        _build_indices(src, dst, batch))

    agg0 = _seg_layer0(x, srcs_plain, dsts, rmeta).reshape(
        M_PAD_NODES, 128)
    h = _mlp0(x, agg0, params['conv0_W1'], params['conv0_b1'],
              params['conv0_W2'], params['conv0_b2'])
    for i in (1, 2):
        agg = _seg_layer(h.reshape(2 * N_NODES, 128), srcs_off, dsts,
                         rmeta)
        h = _mlp(h, agg.reshape(2, M_PAD_NODES, 128),
                 params['conv%d_W1' % i], params['conv%d_b1' % i],
                 params['conv%d_W2' % i], params['conv%d_b2' % i])

    pooled = _seg_pool(h.reshape(2 * N_NODES, 128), pool_srcs, pool_dsts,
                       rmeta)
    pooled = pooled.reshape(2, M_PAD_POOL, 128)[:, :B, :]

    g = jax.random.gumbel(jax.random.key(42), (B, N_PAIRS, 2),
                          dtype=jnp.float32)
    adjflat = _decoder(pooled, params, g[:, :, 0], g[:, :, 1])
    return adjflat[:, :NMAX * NMAX].reshape(B, NMAX, NMAX)


# speedup vs baseline: 1.6220x; 1.5445x over previous
import functools

import numpy as np

import jax
import jax.numpy as jnp
from jax import lax
from jax.experimental import pallas as pl
from jax.experimental.pallas import tpu as pltpu
from jax.experimental.pallas import tpu_sc as plsc

N_NODES = 10000
N_EDGES = 320000
D_IN = 128
HID = 256
LAT = 64
B = 128
NMAX = 50
N_PAIRS = NMAX * (NMAX - 1) // 2

LANES = 128
C_E = 2560
E_PAD = C_E * LANES
M_PAD_NODES = 10240
M_PAD_POOL = 256
C_POOL = 128

_BN = 1.0 + 1e-5


NV = 128 // 16
LBUF = 648
CPS = C_E // 32


def _fold_chunk(rows_v, dst_v, slot, bslot, j, frow, local, lbase,
                carry):
    accs, cur = carry
    for g in range(LANES // 16):
        idvec = dst_v[bslot, j, pl.ds(g * 16, 16)]
        for l in range(16):
            d = idvec[l]
            neq = d != cur

            @pl.when(neq)
            def _flush(accs=accs, cur=cur):
                for v in range(NV):
                    frow[pl.ds(v * 16, 16)] = accs[v]
                pltpu.sync_copy(
                    frow, local.at[pl.ds(lbase + cur * 128, 128)])
            row = [rows_v[slot, g * 16 + l, pl.ds(v * 16, 16)]
                   for v in range(NV)]
            accs = tuple(jnp.where(neq, row[v], accs[v] + row[v])
                         for v in range(NV))
            cur = d
    return accs, cur


def _make_segfold(edge_split, n_ranges_per_tile, lbuf, cps, linear_out,
                  C, MPAD):
    IDXB = min(16, cps)
    NBr = cps // IDXB
    NPAIR = IDXB // 2
    mesh = plsc.VectorSubcoreMesh(core_axis_name="c", subcore_axis_name="s")
    scratch = [
        pltpu.VMEM((2, IDXB, LANES), jnp.int32),
        pltpu.VMEM((2, IDXB, LANES), jnp.int32),
        pltpu.VMEM((2, LANES, 128), jnp.float32),
        pltpu.VMEM_SHARED((16 * lbuf * 128,), jnp.float32),
        pltpu.VMEM((128,), jnp.float32),
        pltpu.VMEM((1024,), jnp.float32),
        pltpu.VMEM((8, LANES), jnp.int32),
        pltpu.SemaphoreType.DMA,
        pltpu.SemaphoreType.DMA,
    ]
    out_rows = MPAD if edge_split else 2 * MPAD

    @functools.partial(
        pl.kernel,
        out_type=jax.ShapeDtypeStruct((out_rows * 128,), jnp.float32),
        mesh=mesh,
        scratch_types=scratch,
    )
    def seg(table, srcs, ldsts, rmeta, out, src_v, dst_v, rows_v, local,
            frow, zbuf, meta_v, gsem, isem):
        c = lax.axis_index("c")
        s = lax.axis_index("s")
        zvec = jnp.zeros((16,), jnp.float32)
        lbase = s * (lbuf * 128)

        def _range_body(rng, _rcarry):
            for zi in range(1024 // 16):
                zbuf[pl.ds(zi * 16, 16)] = zvec
            if edge_split:
                r = c * 16 + s
                base = r * cps
                src_base = base
            elif linear_out:
                r = s
                base = s * cps
                src_base = c * C + base
            else:
                r = 2 * s + rng
                base = r * cps
                src_base = c * C + base

            def zbody(k, carry):
                pltpu.sync_copy(zbuf,
                                local.at[pl.ds(lbase + k * 1024, 1024)])
                return carry
            lax.fori_loop(0, (lbuf * 128) // 1024, zbody, 0)
            pltpu.sync_copy(rmeta.at[pl.ds(r * 8, 8)], meta_v)

            pltpu.sync_copy(srcs.at[pl.ds(src_base, IDXB)], src_v.at[0])
            pltpu.sync_copy(ldsts.at[pl.ds(base, IDXB)], dst_v.at[0])
            pltpu.async_copy(table.at[src_v.at[0, 0]], rows_v.at[0], gsem)

            carry0 = (tuple(zvec for _ in range(NV)),
                      jnp.int32(lbuf - 8))

            def block_body(b, carry):
                bslot = lax.rem(b, 2)
                nbslot = lax.rem(b + 1, 2)

                @pl.when(b + 1 < NBr)
                def _prefetch_idx():
                    pltpu.async_copy(
                        srcs.at[pl.ds(src_base + (b + 1) * IDXB, IDXB)],
                        src_v.at[nbslot], isem)
                    pltpu.async_copy(
                        ldsts.at[pl.ds(base + (b + 1) * IDXB, IDXB)],
                        dst_v.at[nbslot], isem)

                def chunk_body(j, carry2):
                    slot = lax.rem(j, 2)
                    nslot = lax.rem(j + 1, 2)
                    pltpu.make_async_copy(
                        table.at[src_v.at[bslot, j]], rows_v.at[slot],
                        gsem).wait()

                    @pl.when(j + 1 < IDXB)
                    def _fire_next():
                        pltpu.async_copy(
                            table.at[src_v.at[bslot, j + 1]],
                            rows_v.at[nslot], gsem)
                    return _fold_chunk(rows_v, dst_v, slot, bslot, j,
                                       frow, local, lbase, carry2)
                carry = lax.fori_loop(0, IDXB, chunk_body, carry)

                @pl.when(b + 1 < NBr)
                def _next_block():
                    pltpu.make_async_copy(
                        srcs.at[pl.ds(src_base + (b + 1) * IDXB, IDXB)],
                        src_v.at[nbslot], isem).wait()
                    pltpu.make_async_copy(
                        ldsts.at[pl.ds(base + (b + 1) * IDXB, IDXB)],
                        dst_v.at[nbslot], isem).wait()
                    pltpu.async_copy(
                        table.at[src_v.at[nbslot, 0]], rows_v.at[0], gsem)
                return carry
            accs, cur = lax.fori_loop(0, NBr, block_body, carry0)
            for v in range(NV):
                frow[pl.ds(v * 16, 16)] = accs[v]
            pltpu.sync_copy(frow, local.at[pl.ds(lbase + cur * 128, 128)])

            if linear_out:
                pltpu.sync_copy(local.at[pl.ds(lbase, 1024)], zbuf)
                pltpu.sync_copy(
                    zbuf, out.at[pl.ds((c * MPAD + s * 8) * 128, 1024)])
            else:
                mvec = meta_v[0, pl.ds(0, 16)]
                start = mvec[0]
                span = mvec[1]
                ob = (start if edge_split else c * MPAD + start) * 128

                def cbody(k, carry):
                    pltpu.sync_copy(
                        local.at[pl.ds(lbase + k * 1024, 1024)], zbuf)
                    pltpu.sync_copy(zbuf,
                                    out.at[pl.ds(ob + k * 1024, 1024)])
                    return carry
                nfull = span // 8
                lax.fori_loop(0, nfull, cbody, 0)

                def tbody(j, carry):
                    o = nfull * 1024 + j * 128
                    pltpu.sync_copy(
                        local.at[pl.ds(lbase + o, 128)],
                        zbuf.at[pl.ds(0, 128)])
                    pltpu.sync_copy(zbuf.at[pl.ds(0, 128)],
                                    out.at[pl.ds(ob + o, 128)])
                    return carry
                lax.fori_loop(0, span - nfull * 8, tbody, 0)
            return _rcarry
        lax.fori_loop(0, n_ranges_per_tile, _range_body, 0)

    return seg


_seg_layer0 = _make_segfold(True, 1, LBUF, CPS, False, C_E, M_PAD_NODES)
_seg_layer = _make_segfold(False, 2, LBUF, CPS, False, C_E, M_PAD_NODES)
_seg_pool = _make_segfold(False, 1, 16, C_POOL // 16, True, C_POOL,
                          M_PAD_POOL)


def _leaky(t):
    return jnp.where(t >= 0, t, t * 0.2)


def _mlp0_body(x_ref, p_ref, w1_ref, b1_ref, w2_ref, b2_ref, o_ref):
    hin = x_ref[...] + p_ref[...]
    t = jnp.dot(hin, w1_ref[...], preferred_element_type=jnp.float32)
    t = _leaky(t + b1_ref[...])
    t = t / jnp.sqrt(_BN)
    t = jnp.dot(t, w2_ref[...], preferred_element_type=jnp.float32)
    t = _leaky(t + b2_ref[...])
    o_ref[0] = t[:, :128]
    o_ref[1] = t[:, 128:]


def _mlp_body(h_ref, a_ref, w1_ref, b1_ref, w2_ref, b2_ref, o_ref):
    hin = jnp.concatenate(
        [h_ref[0] + a_ref[0], h_ref[1] + a_ref[1]], axis=-1)
    t = jnp.dot(hin, w1_ref[...], preferred_element_type=jnp.float32)
    t = _leaky(t + b1_ref[...])
    t = t / jnp.sqrt(_BN)
    t = jnp.dot(t, w2_ref[...], preferred_element_type=jnp.float32)
    t = _leaky(t + b2_ref[...])
    o_ref[0] = t[:, :128]
    o_ref[1] = t[:, 128:]


_MLP_G = 10
_MLP_R = N_NODES // _MLP_G


def _mlp0(x, parts, W1, b1, W2, b2):
    return pl.pallas_call(
        _mlp0_body,
        grid=(_MLP_G,),
        in_specs=[
            pl.BlockSpec((_MLP_R, 128), lambda g: (g, 0)),
            pl.BlockSpec((_MLP_R, 128), lambda g: (g, 0)),
            pl.BlockSpec((D_IN, HID), lambda g: (0, 0)),
            pl.BlockSpec((1, HID), lambda g: (0, 0)),
            pl.BlockSpec((HID, HID), lambda g: (0, 0)),
            pl.BlockSpec((1, HID), lambda g: (0, 0)),
        ],
        out_specs=pl.BlockSpec((2, _MLP_R, 128), lambda g: (0, g, 0)),
        out_shape=jax.ShapeDtypeStruct((2, N_NODES, 128), jnp.float32),
    )(x, parts, W1, b1.reshape(1, HID), W2, b2.reshape(1, HID))


def _mlp(h, agg, W1, b1, W2, b2):
    return pl.pallas_call(
        _mlp_body,
        grid=(_MLP_G,),
        in_specs=[
            pl.BlockSpec((2, _MLP_R, 128), lambda g: (0, g, 0)),
            pl.BlockSpec((2, _MLP_R, 128), lambda g: (0, g, 0)),
            pl.BlockSpec((HID, HID), lambda g: (0, 0)),
            pl.BlockSpec((1, HID), lambda g: (0, 0)),
            pl.BlockSpec((HID, HID), lambda g: (0, 0)),
            pl.BlockSpec((1, HID), lambda g: (0, 0)),
        ],
        out_specs=pl.BlockSpec((2, _MLP_R, 128), lambda g: (0, g, 0)),
        out_shape=jax.ShapeDtypeStruct((2, N_NODES, 128), jnp.float32),
    )(h, agg, W1, b1.reshape(1, HID), W2, b2.reshape(1, HID))


NPAD_COLS = 1280
ADJ_COLS = 2560


def _build_scatter_mat():
    iu0, iu1 = np.triu_indices(NMAX, k=1)
    S = np.zeros((NPAD_COLS, ADJ_COLS), np.float32)
    p = np.arange(N_PAIRS)
    S[p, iu0 * NMAX + iu1] = 1.0
    S[p, iu1 * NMAX + iu0] = 1.0
    return S


_SCATTER_MAT_NP = _build_scatter_mat()


def _dec_body(p_ref, flw_ref, flb_ref, fmw_ref, fmb_ref, d0w_ref, d0b_ref,
              d1w_ref, d1b_ref, we_ref, be_ref, wo_ref, bo_ref,
              g0_ref, g1_ref, s_ref, o_ref):
    pooled = jnp.concatenate([p_ref[0], p_ref[1]], axis=-1)
    pooled = pooled / jnp.sqrt(_BN)
    zg = jnp.dot(pooled, flw_ref[...],
                 preferred_element_type=jnp.float32) + flb_ref[...]
    mu = jnp.dot(zg, fmw_ref[...],
                 preferred_element_type=jnp.float32) + fmb_ref[...]
    t = jnp.maximum(jnp.dot(mu, d0w_ref[...],
                    preferred_element_type=jnp.float32) + d0b_ref[...], 0.0)
    t = jnp.maximum(jnp.dot(t, d1w_ref[...],
                    preferred_element_type=jnp.float32) + d1b_ref[...], 0.0)
    z0 = jnp.dot(t, we_ref[...],
                 preferred_element_type=jnp.float32) + be_ref[...]
    z1 = jnp.dot(t, wo_ref[...],
                 preferred_element_type=jnp.float32) + bo_ref[...]
    zz = jnp.where(z0 + g0_ref[...] >= z1 + g1_ref[...], 1.0, 0.0)
    o_ref[...] = jnp.dot(zz.astype(jnp.bfloat16), s_ref[...],
                         preferred_element_type=jnp.float32)


def _decoder(pooled, params, g0, g1):
    def pad_cols(a, n):
        return jnp.pad(a, [(0, 0)] * (a.ndim - 1) + [(0, n - a.shape[-1])])

    We = pad_cols(params['dec2_W'][:, 0::2], NPAD_COLS)
    Wo = pad_cols(params['dec2_W'][:, 1::2], NPAD_COLS)
    be = pad_cols(params['dec2_b'][0::2].reshape(1, N_PAIRS), NPAD_COLS)
    bo = pad_cols(params['dec2_b'][1::2].reshape(1, N_PAIRS), NPAD_COLS)
    return pl.pallas_call(
        _dec_body,
        out_shape=jax.ShapeDtypeStruct((B, ADJ_COLS), jnp.float32),
    )(pooled, params['fc_latent_W'], params['fc_latent_b'].reshape(1, LAT),
      params['fc_mu_W'], params['fc_mu_b'].reshape(1, LAT),
      params['dec0_W'], params['dec0_b'].reshape(1, HID),
      params['dec1_W'], params['dec1_b'].reshape(1, HID),
      We, be, Wo, bo, pad_cols(g0, NPAD_COLS), pad_cols(g1, NPAD_COLS),
      jnp.asarray(_SCATTER_MAT_NP, dtype=jnp.bfloat16))


def _build_indices(src, dst, batch):
    order = jnp.argsort(dst, stable=True)
    sdst = dst[order]
    ssrc = src[order]
    RSLOT = CPS * LANES
    cut = sdst[jnp.arange(32, dtype=jnp.int32) * (N_EDGES // 32)]
    start_node = jnp.concatenate([
        jnp.zeros((1,), jnp.int32), cut[1:],
        jnp.full((1,), N_NODES, jnp.int32)])
    estart = jnp.searchsorted(sdst, cut, side='left').astype(jnp.int32)
    eend = jnp.concatenate([estart[1:],
                            jnp.full((1,), N_EDGES, jnp.int32)])
    sidx = jnp.arange(E_PAD, dtype=jnp.int32)
    r = sidx // RSLOT
    p = sidx % RSLOT
    e = estart[r] + p
    valid = e < eend[r]
    ec = jnp.minimum(e, N_EDGES - 1)
    src_p = jnp.where(valid, ssrc[ec], (sidx * 131) % N_NODES)
    ldst = sdst[ec] - start_node[r]
    ldst = jnp.where(ldst < LBUF - 8, ldst, (LBUF - 8) + (sidx % 8))
    dst_p = jnp.where(valid, ldst, (LBUF - 8) + sidx % 8)
    srcs_plain = src_p.reshape(C_E, LANES)
    srcs_off = jnp.concatenate([src_p, src_p + N_NODES]).reshape(
        2 * C_E, LANES)
    dsts = dst_p.reshape(C_E, LANES)
    sn = start_node[:32]
    span = start_node[1:] - start_node[:32]
    rmeta = jnp.zeros((256, LANES), jnp.int32)
    rmeta = rmeta.at[jnp.arange(32) * 8, 0].set(sn)
    rmeta = rmeta.at[jnp.arange(32) * 8, 1].set(span)

    CPP = C_POOL // 32
    PSLOT = CPP * LANES
    gstart = jnp.searchsorted(
        batch, jnp.arange(32, dtype=jnp.int32) * (B // 32),
        side='left').astype(jnp.int32)
    gend = jnp.concatenate([gstart[1:],
                            jnp.full((1,), N_NODES, jnp.int32)])
    qidx = jnp.arange(C_POOL * LANES, dtype=jnp.int32)
    pr = qidx // PSLOT
    pp = qidx % PSLOT
    n = gstart[pr] + pp
    pvalid = n < gend[pr]
    nc = jnp.minimum(n, N_NODES - 1)
    nid = jnp.where(pvalid, n, (qidx * 41) % N_NODES)
    pdst = jnp.where(pvalid, batch[nc] % 8, 8 + qidx % 8)
    pool_srcs = jnp.concatenate([nid, nid + N_NODES]).reshape(
        2 * C_POOL, LANES)
    pool_dsts = pdst.reshape(C_POOL, LANES)
    return srcs_plain, srcs_off, dsts, rmeta, pool_srcs, pool_dsts


def kernel(x, params, edge_index, batch):
    x = x.astype(jnp.float32)
    src = edge_index[0].astype(jnp.int32)
    dst = edge_index[1].astype(jnp.int32)
    batch = batch.astype(jnp.int32)
    srcs_plain, srcs_off, dsts, rmeta, pool_srcs, pool_dsts = (
        _build_indices(src, dst, batch))

    agg0 = _seg_layer0(x, srcs_plain, dsts, rmeta).reshape(
        M_PAD_NODES, 128)
    h = _mlp0(x, agg0, params['conv0_W1'], params['conv0_b1'],
              params['conv0_W2'], params['conv0_b2'])
    for i in (1, 2):
        agg = _seg_layer(h.reshape(2 * N_NODES, 128), srcs_off, dsts,
                         rmeta)
        h = _mlp(h, agg.reshape(2, M_PAD_NODES, 128),
                 params['conv%d_W1' % i], params['conv%d_b1' % i],
                 params['conv%d_W2' % i], params['conv%d_b2' % i])

    pooled = _seg_pool(h.reshape(2 * N_NODES, 128), pool_srcs, pool_dsts,
                       rmeta)
    pooled = pooled.reshape(2, M_PAD_POOL, 128)[:, :B, :]

    g = jax.random.gumbel(jax.random.key(42), (B, N_PAIRS, 2),
                          dtype=jnp.float32)
    adjflat = _decoder(pooled, params, g[:, :, 0], g[:, :, 1])
    return adjflat[:, :NMAX * NMAX].reshape(B, NMAX, NMAX)


# speedup vs baseline: 1.6583x; 1.0224x over previous
import functools

import numpy as np

import jax
import jax.numpy as jnp
from jax import lax
from jax.experimental import pallas as pl
from jax.experimental.pallas import tpu as pltpu
from jax.experimental.pallas import tpu_sc as plsc

N_NODES = 10000
N_EDGES = 320000
D_IN = 128
HID = 256
LAT = 64
B = 128
NMAX = 50
N_PAIRS = NMAX * (NMAX - 1) // 2

LANES = 128
C_E = 2560
E_PAD = C_E * LANES
M_PAD_NODES = 10240
M_PAD_POOL = 256
C_POOL = 128

_BN = 1.0 + 1e-5


NV = 128 // 16
LBUF = 648
CPS = C_E // 32


def _fold_chunk(rows_v, dst_v, slot, bslot, j, frow, local, lbase,
                carry):
    accs, cur = carry
    for g in range(LANES // 16):
        idvec = dst_v[bslot, j, pl.ds(g * 16, 16)]
        for l in range(16):
            d = idvec[l]
            neq = d != cur

            @pl.when(neq)
            def _flush(accs=accs, cur=cur):
                for v in range(NV):
                    frow[pl.ds(v * 16, 16)] = accs[v]
                pltpu.sync_copy(
                    frow, local.at[pl.ds(lbase + cur * 128, 128)])
            row = [rows_v[slot, g * 16 + l, pl.ds(v * 16, 16)]
                   for v in range(NV)]
            accs = tuple(jnp.where(neq, row[v], accs[v] + row[v])
                         for v in range(NV))
            cur = d
    return accs, cur


def _make_segfold(edge_split, n_ranges_per_tile, lbuf, cps, linear_out,
                  C, MPAD):
    IDXB = min(16, cps)
    NBr = cps // IDXB
    NPAIR = IDXB // 2
    mesh = plsc.VectorSubcoreMesh(core_axis_name="c", subcore_axis_name="s")
    scratch = [
        pltpu.VMEM((2, IDXB, LANES), jnp.int32),
        pltpu.VMEM((2, IDXB, LANES), jnp.int32),
        pltpu.VMEM((2, LANES, 128), jnp.float32),
        pltpu.VMEM_SHARED((16 * lbuf * 128,), jnp.float32),
        pltpu.VMEM((128,), jnp.float32),
        pltpu.VMEM((1024,), jnp.float32),
        pltpu.VMEM((8, LANES), jnp.int32),
        pltpu.SemaphoreType.DMA,
        pltpu.SemaphoreType.DMA,
    ]
    out_rows = MPAD if edge_split else 2 * MPAD

    @functools.partial(
        pl.kernel,
        out_type=jax.ShapeDtypeStruct((out_rows * 128,), jnp.float32),
        mesh=mesh,
        scratch_types=scratch,
    )
    def seg(table, srcs, ldsts, rmeta, out, src_v, dst_v, rows_v, local,
            frow, zbuf, meta_v, gsem, isem):
        c = lax.axis_index("c")
        s = lax.axis_index("s")
        zvec = jnp.zeros((16,), jnp.float32)
        lbase = s * (lbuf * 128)

        def _range_body(rng, _rcarry):
            for zi in range(1024 // 16):
                zbuf[pl.ds(zi * 16, 16)] = zvec
            if edge_split:
                r = c * 16 + s
                base = r * cps
                src_base = base
            elif linear_out:
                r = s
                base = s * cps
                src_base = c * C + base
            else:
                r = 2 * s + rng
                base = r * cps
                src_base = c * C + base

            def zbody(k, carry):
                pltpu.sync_copy(zbuf,
                                local.at[pl.ds(lbase + k * 1024, 1024)])
                return carry
            lax.fori_loop(0, (lbuf * 128) // 1024, zbody, 0)
            pltpu.sync_copy(rmeta.at[pl.ds(r * 8, 8)], meta_v)

            pltpu.sync_copy(srcs.at[pl.ds(src_base, IDXB)], src_v.at[0])
            pltpu.sync_copy(ldsts.at[pl.ds(base, IDXB)], dst_v.at[0])
            pltpu.async_copy(table.at[src_v.at[0, 0]], rows_v.at[0], gsem)

            carry0 = (tuple(zvec for _ in range(NV)),
                      jnp.int32(lbuf - 8))

            def block_body(b, carry):
                bslot = lax.rem(b, 2)
                nbslot = lax.rem(b + 1, 2)

                @pl.when(b + 1 < NBr)
                def _prefetch_idx():
                    pltpu.async_copy(
                        srcs.at[pl.ds(src_base + (b + 1) * IDXB, IDXB)],
                        src_v.at[nbslot], isem)
                    pltpu.async_copy(
                        ldsts.at[pl.ds(base + (b + 1) * IDXB, IDXB)],
                        dst_v.at[nbslot], isem)

                def chunk_body(j, carry2):
                    slot = lax.rem(j, 2)
                    nslot = lax.rem(j + 1, 2)
                    pltpu.make_async_copy(
                        table.at[src_v.at[bslot, j]], rows_v.at[slot],
                        gsem).wait()

                    @pl.when(j + 1 < IDXB)
                    def _fire_next():
                        pltpu.async_copy(
                            table.at[src_v.at[bslot, j + 1]],
                            rows_v.at[nslot], gsem)
                    return _fold_chunk(rows_v, dst_v, slot, bslot, j,
                                       frow, local, lbase, carry2)
                carry = lax.fori_loop(0, IDXB, chunk_body, carry)

                @pl.when(b + 1 < NBr)
                def _next_block():
                    pltpu.make_async_copy(
                        srcs.at[pl.ds(src_base + (b + 1) * IDXB, IDXB)],
                        src_v.at[nbslot], isem).wait()
                    pltpu.make_async_copy(
                        ldsts.at[pl.ds(base + (b + 1) * IDXB, IDXB)],
                        dst_v.at[nbslot], isem).wait()
                    pltpu.async_copy(
                        table.at[src_v.at[nbslot, 0]], rows_v.at[0], gsem)
                return carry
            accs, cur = lax.fori_loop(0, NBr, block_body, carry0)
            for v in range(NV):
                frow[pl.ds(v * 16, 16)] = accs[v]
            pltpu.sync_copy(frow, local.at[pl.ds(lbase + cur * 128, 128)])

            if linear_out:
                pltpu.sync_copy(local.at[pl.ds(lbase, 1024)], zbuf)
                pltpu.sync_copy(
                    zbuf, out.at[pl.ds((c * MPAD + s * 8) * 128, 1024)])
            else:
                mvec = meta_v[0, pl.ds(0, 16)]
                start = mvec[0]
                span = mvec[1]
                ob = (start if edge_split else c * MPAD + start) * 128

                def cbody(k, carry):
                    pltpu.sync_copy(
                        local.at[pl.ds(lbase + k * 1024, 1024)], zbuf)
                    pltpu.sync_copy(zbuf,
                                    out.at[pl.ds(ob + k * 1024, 1024)])
                    return carry
                nfull = span // 8
                lax.fori_loop(0, nfull, cbody, 0)

                def tbody(j, carry):
                    o = nfull * 1024 + j * 128
                    pltpu.sync_copy(
                        local.at[pl.ds(lbase + o, 128)],
                        zbuf.at[pl.ds(0, 128)])
                    pltpu.sync_copy(zbuf.at[pl.ds(0, 128)],
                                    out.at[pl.ds(ob + o, 128)])
                    return carry
                lax.fori_loop(0, span - nfull * 8, tbody, 0)
            return _rcarry
        lax.fori_loop(0, n_ranges_per_tile, _range_body, 0)

    return seg


_seg_layer0 = _make_segfold(True, 1, LBUF, CPS, False, C_E, M_PAD_NODES)
_seg_layer = _make_segfold(False, 2, LBUF, CPS, False, C_E, M_PAD_NODES)
_seg_pool = _make_segfold(False, 1, 16, C_POOL // 16, True, C_POOL,
                          M_PAD_POOL)


def _leaky(t):
    return jnp.where(t >= 0, t, t * 0.2)


def _mlp0_body(x_ref, p_ref, w1_ref, b1_ref, w2_ref, b2_ref, o_ref):
    hin = x_ref[...] + p_ref[...]
    t = jnp.dot(hin, w1_ref[...], preferred_element_type=jnp.float32)
    t = _leaky(t + b1_ref[...])
    t = t / jnp.sqrt(_BN)
    t = jnp.dot(t, w2_ref[...], preferred_element_type=jnp.float32)
    t = _leaky(t + b2_ref[...])
    o_ref[0] = t[:, :128]
    o_ref[1] = t[:, 128:]


def _mlp_body(h_ref, a_ref, w1_ref, b1_ref, w2_ref, b2_ref, o_ref):
    hin = jnp.concatenate(
        [h_ref[0] + a_ref[0], h_ref[1] + a_ref[1]], axis=-1)
    t = jnp.dot(hin, w1_ref[...], preferred_element_type=jnp.float32)
    t = _leaky(t + b1_ref[...])
    t = t / jnp.sqrt(_BN)
    t = jnp.dot(t, w2_ref[...], preferred_element_type=jnp.float32)
    t = _leaky(t + b2_ref[...])
    o_ref[0] = t[:, :128]
    o_ref[1] = t[:, 128:]


_MLP_G = 10
_MLP_R = N_NODES // _MLP_G


def _mlp0(x, parts, W1, b1, W2, b2):
    return pl.pallas_call(
        _mlp0_body,
        grid=(_MLP_G,),
        in_specs=[
            pl.BlockSpec((_MLP_R, 128), lambda g: (g, 0)),
            pl.BlockSpec((_MLP_R, 128), lambda g: (g, 0)),
            pl.BlockSpec((D_IN, HID), lambda g: (0, 0)),
            pl.BlockSpec((1, HID), lambda g: (0, 0)),
            pl.BlockSpec((HID, HID), lambda g: (0, 0)),
            pl.BlockSpec((1, HID), lambda g: (0, 0)),
        ],
        out_specs=pl.BlockSpec((2, _MLP_R, 128), lambda g: (0, g, 0)),
        out_shape=jax.ShapeDtypeStruct((2, N_NODES, 128), jnp.float32),
    )(x, parts, W1, b1.reshape(1, HID), W2, b2.reshape(1, HID))


def _mlp(h, agg, W1, b1, W2, b2):
    return pl.pallas_call(
        _mlp_body,
        grid=(_MLP_G,),
        in_specs=[
            pl.BlockSpec((2, _MLP_R, 128), lambda g: (0, g, 0)),
            pl.BlockSpec((2, _MLP_R, 128), lambda g: (0, g, 0)),
            pl.BlockSpec((HID, HID), lambda g: (0, 0)),
            pl.BlockSpec((1, HID), lambda g: (0, 0)),
            pl.BlockSpec((HID, HID), lambda g: (0, 0)),
            pl.BlockSpec((1, HID), lambda g: (0, 0)),
        ],
        out_specs=pl.BlockSpec((2, _MLP_R, 128), lambda g: (0, g, 0)),
        out_shape=jax.ShapeDtypeStruct((2, N_NODES, 128), jnp.float32),
    )(h, agg, W1, b1.reshape(1, HID), W2, b2.reshape(1, HID))


NPAD_COLS = 1280
ADJ_COLS = 2560


def _build_scatter_mat():
    iu0, iu1 = np.triu_indices(NMAX, k=1)
    S = np.zeros((NPAD_COLS, ADJ_COLS), np.float32)
    p = np.arange(N_PAIRS)
    S[p, iu0 * NMAX + iu1] = 1.0
    S[p, iu1 * NMAX + iu0] = 1.0
    return S


_SCATTER_MAT_NP = _build_scatter_mat()


def _dec_body(p_ref, flw_ref, flb_ref, fmw_ref, fmb_ref, d0w_ref, d0b_ref,
              d1w_ref, d1b_ref, we_ref, be_ref, wo_ref, bo_ref,
              g0_ref, g1_ref, s_ref, o_ref):
    pooled = jnp.concatenate([p_ref[0], p_ref[1]], axis=-1)
    pooled = pooled / jnp.sqrt(_BN)
    zg = jnp.dot(pooled, flw_ref[...],
                 preferred_element_type=jnp.float32) + flb_ref[...]
    mu = jnp.dot(zg, fmw_ref[...],
                 preferred_element_type=jnp.float32) + fmb_ref[...]
    t = jnp.maximum(jnp.dot(mu, d0w_ref[...],
                    preferred_element_type=jnp.float32) + d0b_ref[...], 0.0)
    t = jnp.maximum(jnp.dot(t, d1w_ref[...],
                    preferred_element_type=jnp.float32) + d1b_ref[...], 0.0)
    z0 = jnp.dot(t, we_ref[...],
                 preferred_element_type=jnp.float32) + be_ref[...]
    z1 = jnp.dot(t, wo_ref[...],
                 preferred_element_type=jnp.float32) + bo_ref[...]
    zz = jnp.where(z0 + g0_ref[...] >= z1 + g1_ref[...], 1.0, 0.0)
    o_ref[...] = jnp.dot(zz.astype(jnp.bfloat16), s_ref[...],
                         preferred_element_type=jnp.float32)


def _decoder(pooled, params, g0, g1):
    def pad_cols(a, n):
        return jnp.pad(a, [(0, 0)] * (a.ndim - 1) + [(0, n - a.shape[-1])])

    We = pad_cols(params['dec2_W'][:, 0::2], NPAD_COLS)
    Wo = pad_cols(params['dec2_W'][:, 1::2], NPAD_COLS)
    be = pad_cols(params['dec2_b'][0::2].reshape(1, N_PAIRS), NPAD_COLS)
    bo = pad_cols(params['dec2_b'][1::2].reshape(1, N_PAIRS), NPAD_COLS)
    return pl.pallas_call(
        _dec_body,
        out_shape=jax.ShapeDtypeStruct((B, ADJ_COLS), jnp.float32),
    )(pooled, params['fc_latent_W'], params['fc_latent_b'].reshape(1, LAT),
      params['fc_mu_W'], params['fc_mu_b'].reshape(1, LAT),
      params['dec0_W'], params['dec0_b'].reshape(1, HID),
      params['dec1_W'], params['dec1_b'].reshape(1, HID),
      We, be, Wo, bo, pad_cols(g0, NPAD_COLS), pad_cols(g1, NPAD_COLS),
      jnp.asarray(_SCATTER_MAT_NP, dtype=jnp.bfloat16))


def _build_indices(src, dst, batch):
    sdst, ssrc = lax.sort((dst, src), num_keys=1, is_stable=True)
    RSLOT = CPS * LANES
    cut = sdst[jnp.arange(32, dtype=jnp.int32) * (N_EDGES // 32)]
    start_node = jnp.concatenate([
        jnp.zeros((1,), jnp.int32), cut[1:],
        jnp.full((1,), N_NODES, jnp.int32)])
    estart = jnp.searchsorted(sdst, cut, side='left').astype(jnp.int32)
    eend = jnp.concatenate([estart[1:],
                            jnp.full((1,), N_EDGES, jnp.int32)])
    sidx = jnp.arange(E_PAD, dtype=jnp.int32)
    r = sidx // RSLOT
    p = sidx % RSLOT
    e = estart[r] + p
    valid = e < eend[r]
    ec = jnp.minimum(e, N_EDGES - 1)
    src_p = jnp.where(valid, ssrc[ec], (sidx * 131) % N_NODES)
    ldst = sdst[ec] - start_node[r]
    ldst = jnp.where(ldst < LBUF - 8, ldst, (LBUF - 8) + (sidx % 8))
    dst_p = jnp.where(valid, ldst, (LBUF - 8) + sidx % 8)
    srcs_plain = src_p.reshape(C_E, LANES)
    srcs_off = jnp.concatenate([src_p, src_p + N_NODES]).reshape(
        2 * C_E, LANES)
    dsts = dst_p.reshape(C_E, LANES)
    sn = start_node[:32]
    span = start_node[1:] - start_node[:32]
    rmeta = jnp.zeros((256, LANES), jnp.int32)
    rmeta = rmeta.at[jnp.arange(32) * 8, 0].set(sn)
    rmeta = rmeta.at[jnp.arange(32) * 8, 1].set(span)

    CPP = C_POOL // 32
    PSLOT = CPP * LANES
    gstart = jnp.searchsorted(
        batch, jnp.arange(32, dtype=jnp.int32) * (B // 32),
        side='left').astype(jnp.int32)
    gend = jnp.concatenate([gstart[1:],
                            jnp.full((1,), N_NODES, jnp.int32)])
    qidx = jnp.arange(C_POOL * LANES, dtype=jnp.int32)
    pr = qidx // PSLOT
    pp = qidx % PSLOT
    n = gstart[pr] + pp
    pvalid = n < gend[pr]
    nc = jnp.minimum(n, N_NODES - 1)
    nid = jnp.where(pvalid, n, (qidx * 41) % N_NODES)
    pdst = jnp.where(pvalid, batch[nc] % 8, 8 + qidx % 8)
    pool_srcs = jnp.concatenate([nid, nid + N_NODES]).reshape(
        2 * C_POOL, LANES)
    pool_dsts = pdst.reshape(C_POOL, LANES)
    return srcs_plain, srcs_off, dsts, rmeta, pool_srcs, pool_dsts


def kernel(x, params, edge_index, batch):
    x = x.astype(jnp.float32)
    src = edge_index[0].astype(jnp.int32)
    dst = edge_index[1].astype(jnp.int32)
    batch = batch.astype(jnp.int32)
    srcs_plain, srcs_off, dsts, rmeta, pool_srcs, pool_dsts = (
        _build_indices(src, dst, batch))

    agg0 = _seg_layer0(x, srcs_plain, dsts, rmeta).reshape(
        M_PAD_NODES, 128)
    h = _mlp0(x, agg0, params['conv0_W1'], params['conv0_b1'],
              params['conv0_W2'], params['conv0_b2'])
    for i in (1, 2):
        agg = _seg_layer(h.reshape(2 * N_NODES, 128), srcs_off, dsts,
                         rmeta)
        h = _mlp(h, agg.reshape(2, M_PAD_NODES, 128),
                 params['conv%d_W1' % i], params['conv%d_b1' % i],
                 params['conv%d_W2' % i], params['conv%d_b2' % i])

    pooled = _seg_pool(h.reshape(2 * N_NODES, 128), pool_srcs, pool_dsts,
                       rmeta)
    pooled = pooled.reshape(2, M_PAD_POOL, 128)[:, :B, :]

    g = jax.random.gumbel(jax.random.key(42), (B, N_PAIRS, 2),
                          dtype=jnp.float32)
    adjflat = _decoder(pooled, params, g[:, :, 0], g[:, :, 1])
    return adjflat[:, :NMAX * NMAX].reshape(B, NMAX, NMAX)
